# Initial kernel scaffold; baseline (speedup 1.0000x reference)
import jax
import jax.numpy as jnp
from jax.experimental import pallas as pl


def kernel(position_ids, pos_enc):
    raise NotImplementedError("write your pallas kernel here")


# speedup vs baseline: 1.9803x; 1.9803x over previous
import functools

import jax
import jax.numpy as jnp
from jax import lax
from jax.experimental import pallas as pl
from jax.experimental.pallas import tpu as pltpu
from jax.experimental.pallas import tpu_sc as plsc

HIDDEN = 1024
NC = 2
NS = 16
NW = NC * NS
K = 32


def _gather_body(table_hbm, idx_hbm, out_hbm, idx_v, rows_v, gsem, b_per_w,
                 n_chunks):
    wid = lax.axis_index("s") * NC + lax.axis_index("c")
    base = wid * b_per_w
    pltpu.sync_copy(idx_hbm.at[pl.ds(base, b_per_w)], idx_v)

    def chunk_body(c, carry):
        pltpu.async_copy(
            table_hbm.at[idx_v.at[pl.ds(c * K, K)]], rows_v, gsem
        ).wait()
        pltpu.sync_copy(rows_v, out_hbm.at[pl.ds(base + c * K, K)])
        return carry

    lax.fori_loop(0, n_chunks, chunk_body, 0)


def kernel(position_ids, pos_enc):
    orig_shape = position_ids.shape
    idx_flat = jnp.reshape(position_ids, (-1,)).astype(jnp.int32)
    B = idx_flat.shape[0]
    b_per_w = B // NW
    n_chunks = b_per_w // K

    mesh = plsc.VectorSubcoreMesh(core_axis_name="c", subcore_axis_name="s")
    body = functools.partial(_gather_body, b_per_w=b_per_w, n_chunks=n_chunks)
    out = pl.kernel(
        body,
        out_type=jax.ShapeDtypeStruct((B, HIDDEN), jnp.float32),
        mesh=mesh,
        scratch_types=[
            pltpu.VMEM((b_per_w,), jnp.int32),
            pltpu.VMEM((K, HIDDEN), jnp.float32),
            pltpu.SemaphoreType.DMA,
        ],
    )(pos_enc, idx_flat)
    return jnp.reshape(out, orig_shape + (HIDDEN,))


# speedup vs baseline: 2.3642x; 1.1939x over previous
import functools

import jax
import jax.numpy as jnp
from jax import lax
from jax.experimental import pallas as pl
from jax.experimental.pallas import tpu as pltpu
from jax.experimental.pallas import tpu_sc as plsc

HIDDEN = 1024
NC = 2
NS = 16
NW = NC * NS
K = 32
NBUF = 2


def _gather_body(table_hbm, idx_hbm, out_hbm, idx_v, rows_v, gsems, wsems,
                 b_per_w, n_chunks):
    wid = lax.axis_index("s") * NC + lax.axis_index("c")
    base = wid * b_per_w
    pltpu.sync_copy(idx_hbm.at[pl.ds(base, b_per_w)], idx_v)

    def start_gather(chunk, b):
        pltpu.async_copy(
            table_hbm.at[idx_v.at[pl.ds(chunk * K, K)]], rows_v.at[b],
            gsems[b])

    for b in range(NBUF):
        start_gather(b, b)

    def chunk_group(g, carry):
        c = g * NBUF
        for b in range(NBUF):
            chunk = c + b
            pltpu.make_async_copy(
                table_hbm.at[idx_v.at[pl.ds(0, K)]], rows_v.at[b],
                gsems[b]).wait()
            pltpu.async_copy(
                rows_v.at[b], out_hbm.at[pl.ds(base + chunk * K, K)],
                wsems[b])
            pltpu.make_async_copy(
                rows_v.at[b], out_hbm.at[pl.ds(0, K)], wsems[b]).wait()
            @pl.when(chunk + NBUF < n_chunks)
            def _():
                start_gather(chunk + NBUF, b)
        return carry

    lax.fori_loop(0, n_chunks // NBUF, chunk_group, 0)


def kernel(position_ids, pos_enc):
    orig_shape = position_ids.shape
    idx_flat = jnp.reshape(position_ids, (-1,)).astype(jnp.int32)
    B = idx_flat.shape[0]
    b_per_w = B // NW
    n_chunks = b_per_w // K
    assert n_chunks % NBUF == 0

    mesh = plsc.VectorSubcoreMesh(core_axis_name="c", subcore_axis_name="s")
    body = functools.partial(_gather_body, b_per_w=b_per_w, n_chunks=n_chunks)
    out = pl.kernel(
        body,
        out_type=jax.ShapeDtypeStruct((B, HIDDEN), jnp.float32),
        mesh=mesh,
        scratch_types=[
            pltpu.VMEM((b_per_w,), jnp.int32),
            pltpu.VMEM((NBUF, K, HIDDEN), jnp.float32),
            [pltpu.SemaphoreType.DMA] * NBUF,
            [pltpu.SemaphoreType.DMA] * NBUF,
        ],
    )(pos_enc, idx_flat)
    return jnp.reshape(out, orig_shape + (HIDDEN,))


# speedup vs baseline: 2.3661x; 1.0008x over previous
import functools

import jax
import jax.numpy as jnp
from jax import lax
from jax.experimental import pallas as pl
from jax.experimental.pallas import tpu as pltpu
from jax.experimental.pallas import tpu_sc as plsc

HIDDEN = 1024
NC = 2
NS = 16
NW = NC * NS
K = 16
NBUF = 4
DEPTH = 2


def _gather_body(table_hbm, idx_hbm, out_hbm, idx_v, rows_v, gsems, wsems,
                 b_per_w, n_chunks):
    wid = lax.axis_index("s") * NC + lax.axis_index("c")
    base = wid * b_per_w
    pltpu.sync_copy(idx_hbm.at[pl.ds(base, b_per_w)], idx_v)

    def start_gather(chunk, b):
        pltpu.async_copy(
            table_hbm.at[idx_v.at[pl.ds(chunk * K, K)]], rows_v.at[b],
            gsems[b])

    for c in range(DEPTH):
        start_gather(c, c % NBUF)

    def chunk_group(g, carry):
        c0 = g * NBUF
        for j in range(NBUF):
            chunk = c0 + j
            pltpu.make_async_copy(
                table_hbm.at[idx_v.at[pl.ds(0, K)]], rows_v.at[j],
                gsems[j]).wait()
            pltpu.async_copy(
                rows_v.at[j], out_hbm.at[pl.ds(base + chunk * K, K)],
                wsems[j])
            nb = (j + DEPTH) % NBUF
            @pl.when(chunk + DEPTH < n_chunks)
            def _():
                @pl.when(chunk + DEPTH >= NBUF)
                def _():
                    pltpu.make_async_copy(
                        rows_v.at[nb], out_hbm.at[pl.ds(0, K)],
                        wsems[nb]).wait()
                start_gather(chunk + DEPTH, nb)
        return carry

    lax.fori_loop(0, n_chunks // NBUF, chunk_group, 0)
    for b in range(NBUF):
        pltpu.make_async_copy(
            rows_v.at[b], out_hbm.at[pl.ds(0, K)], wsems[b]).wait()


def kernel(position_ids, pos_enc):
    orig_shape = position_ids.shape
    idx_flat = jnp.reshape(position_ids, (-1,)).astype(jnp.int32)
    B = idx_flat.shape[0]
    b_per_w = B // NW
    n_chunks = b_per_w // K
    assert n_chunks % NBUF == 0

    mesh = plsc.VectorSubcoreMesh(core_axis_name="c", subcore_axis_name="s")
    body = functools.partial(_gather_body, b_per_w=b_per_w, n_chunks=n_chunks)
    out = pl.kernel(
        body,
        out_type=jax.ShapeDtypeStruct((B, HIDDEN), jnp.float32),
        mesh=mesh,
        scratch_types=[
            pltpu.VMEM((b_per_w,), jnp.int32),
            pltpu.VMEM((NBUF, K, HIDDEN), jnp.float32),
            [pltpu.SemaphoreType.DMA] * NBUF,
            [pltpu.SemaphoreType.DMA] * NBUF,
        ],
    )(pos_enc, idx_flat)
    return jnp.reshape(out, orig_shape + (HIDDEN,))
